# baseline (speedup 1.0000x reference)
import jax
import jax.numpy as jnp
from jax import lax
from jax.experimental import pallas as pl
from jax.experimental.pallas import tpu as pltpu

_KI = 8
_KO = 4


def _dot_t(a, b):
    return lax.dot_general(a, b, (((0,), (0,)), ((), ())),
                           preferred_element_type=jnp.float32)


def _dot_nt(a, b):
    return lax.dot_general(a, b, (((1,), (1,)), ((), ())),
                           preferred_element_type=jnp.float32)


def _hgmp_kernel(x_hbm, inc_hbm, w_ref, b_ref, eps_ref, out_hbm,
                 x_sc, inc_sc, y_sc, sx, sinc, sy):
    n, d = x_sc.shape
    ri = n // _KI
    ro = n // _KO

    cpx = pltpu.make_async_copy(x_hbm, x_sc, sx)
    cpx.start()
    inc_cps = []
    for i in range(_KI):
        cp = pltpu.make_async_copy(
            inc_hbm.at[pl.ds(i * ri, ri), :],
            inc_sc.at[pl.ds(i * ri, ri), :],
            sinc.at[i])
        cp.start()
        inc_cps.append(cp)

    cpx.wait()
    x = x_sc[...]
    w = w_ref[...]
    xwb = (1.0 + eps_ref[0, 0]) * _dot_nt(x, w) + b_ref[...]

    sums = jnp.zeros((inc_sc.shape[1], d), jnp.float32)
    counts = jnp.zeros((inc_sc.shape[1], 1), jnp.float32)
    ones = jnp.ones((ri, 1), dtype=jnp.float32)
    for i in range(_KI):
        inc_cps[i].wait()
        m = (inc_sc[pl.ds(i * ri, ri), :] > 0).astype(jnp.float32)
        sums = sums + _dot_t(m, x[i * ri:(i + 1) * ri, :])
        counts = counts + _dot_t(m, ones)

    vf = sums / jnp.maximum(counts, 1.0)
    vfw = _dot_nt(vf, w)

    out_cps = []
    for j in range(_KO):
        m = (inc_sc[pl.ds(j * ro, ro), :] > 0).astype(jnp.float32)
        y = xwb[j * ro:(j + 1) * ro, :] + jnp.dot(
            m, vfw, preferred_element_type=jnp.float32)
        y_sc[pl.ds(j * ro, ro), :] = jnp.where(y > 0, y, 0.2 * y)
        cp = pltpu.make_async_copy(
            y_sc.at[pl.ds(j * ro, ro), :],
            out_hbm.at[pl.ds(j * ro, ro), :],
            sy.at[j])
        cp.start()
        out_cps.append(cp)
    for cp in out_cps:
        cp.wait()


def kernel(node_features, incidence_matrix, W, b, epsilon):
    N, D = node_features.shape
    V = incidence_matrix.shape[1]
    b2 = b.reshape(1, D)
    eps2 = epsilon.reshape(1, 1)
    return pl.pallas_call(
        _hgmp_kernel,
        in_specs=[
            pl.BlockSpec(memory_space=pl.ANY),
            pl.BlockSpec(memory_space=pl.ANY),
            pl.BlockSpec((D, D), lambda: (0, 0)),
            pl.BlockSpec((1, D), lambda: (0, 0)),
            pl.BlockSpec((1, 1), lambda: (0, 0)),
        ],
        out_specs=pl.BlockSpec(memory_space=pl.ANY),
        out_shape=jax.ShapeDtypeStruct((N, D), jnp.float32),
        scratch_shapes=[
            pltpu.VMEM((N, D), jnp.float32),
            pltpu.VMEM((N, V), jnp.float32),
            pltpu.VMEM((N, D), jnp.float32),
            pltpu.SemaphoreType.DMA,
            pltpu.SemaphoreType.DMA((_KI,)),
            pltpu.SemaphoreType.DMA((_KO,)),
        ],
    )(node_features, incidence_matrix, W, b2, eps2)


# speedup vs baseline: 1.0656x; 1.0656x over previous
import jax
import jax.numpy as jnp
from jax import lax
from jax.experimental import pallas as pl
from jax.experimental.pallas import tpu as pltpu

_KO = 4


def _dot_t(a, b):
    return lax.dot_general(a, b, (((0,), (0,)), ((), ())),
                           preferred_element_type=jnp.float32)


def _dot_nt(a, b):
    return lax.dot_general(a, b, (((1,), (1,)), ((), ())),
                           preferred_element_type=jnp.float32)


def _hgmp_kernel(x_hbm, inc_hbm, w_ref, b_ref, eps_ref, out_hbm,
                 x_sc, inc_sc, y_sc, sx, sinc, sy):
    n, d = x_sc.shape
    ro = n // _KO

    cpi = pltpu.make_async_copy(inc_hbm, inc_sc, sinc)
    cpi.start()
    cpx = pltpu.make_async_copy(x_hbm, x_sc, sx)
    cpx.start()

    cpx.wait()
    x = x_sc[...]
    w = w_ref[...]
    xwb = (1.0 + eps_ref[0, 0]) * _dot_nt(x, w) + b_ref[...]

    cpi.wait()
    m = (inc_sc[...] > 0).astype(jnp.float32)
    sums = _dot_t(m, x)
    ones = jnp.ones((n, 1), dtype=jnp.float32)
    counts = _dot_t(m, ones)
    vf = sums / jnp.maximum(counts, 1.0)
    vfw = _dot_nt(vf, w)

    out_cps = []
    for j in range(_KO):
        sl = slice(j * ro, (j + 1) * ro)
        y = xwb[sl, :] + jnp.dot(m[sl, :], vfw,
                                 preferred_element_type=jnp.float32)
        y_sc[pl.ds(j * ro, ro), :] = jnp.where(y > 0, y, 0.2 * y)
        cp = pltpu.make_async_copy(
            y_sc.at[pl.ds(j * ro, ro), :],
            out_hbm.at[pl.ds(j * ro, ro), :],
            sy.at[j])
        cp.start()
        out_cps.append(cp)
    for cp in out_cps:
        cp.wait()


def kernel(node_features, incidence_matrix, W, b, epsilon):
    N, D = node_features.shape
    V = incidence_matrix.shape[1]
    b2 = b.reshape(1, D)
    eps2 = epsilon.reshape(1, 1)
    return pl.pallas_call(
        _hgmp_kernel,
        in_specs=[
            pl.BlockSpec(memory_space=pl.ANY),
            pl.BlockSpec(memory_space=pl.ANY),
            pl.BlockSpec((D, D), lambda: (0, 0)),
            pl.BlockSpec((1, D), lambda: (0, 0)),
            pl.BlockSpec((1, 1), lambda: (0, 0)),
        ],
        out_specs=pl.BlockSpec(memory_space=pl.ANY),
        out_shape=jax.ShapeDtypeStruct((N, D), jnp.float32),
        scratch_shapes=[
            pltpu.VMEM((N, D), jnp.float32),
            pltpu.VMEM((N, V), jnp.float32),
            pltpu.VMEM((N, D), jnp.float32),
            pltpu.SemaphoreType.DMA,
            pltpu.SemaphoreType.DMA,
            pltpu.SemaphoreType.DMA((_KO,)),
        ],
    )(node_features, incidence_matrix, W, b2, eps2)


# speedup vs baseline: 1.1548x; 1.0837x over previous
import jax
import jax.numpy as jnp
from jax import lax
from jax.experimental import pallas as pl
from jax.experimental.pallas import tpu as pltpu


def _dot_t(a, b):
    return lax.dot_general(a, b, (((0,), (0,)), ((), ())),
                           preferred_element_type=jnp.float32)


def _dot_nt(a, b):
    return lax.dot_general(a, b, (((1,), (1,)), ((), ())),
                           preferred_element_type=jnp.float32)


def _hgmp_kernel(x_ref, inc_hbm, w_ref, b_ref, eps_ref, out_ref, inc_sc, sinc):
    n = x_ref.shape[0]
    cpi = pltpu.make_async_copy(inc_hbm, inc_sc, sinc)
    cpi.start()

    x = x_ref[...]
    w = w_ref[...]
    xwb = (1.0 + eps_ref[0, 0]) * _dot_nt(x, w) + b_ref[...]

    cpi.wait()
    m = (inc_sc[...] > 0).astype(jnp.float32)
    sums = _dot_t(m, x)
    ones = jnp.ones((n, 1), dtype=jnp.float32)
    counts = _dot_t(m, ones)
    vf = sums / jnp.maximum(counts, 1.0)
    vfw = _dot_nt(vf, w)

    y = xwb + jnp.dot(m, vfw, preferred_element_type=jnp.float32)
    out_ref[...] = jnp.where(y > 0, y, 0.2 * y)


def kernel(node_features, incidence_matrix, W, b, epsilon):
    N, D = node_features.shape
    V = incidence_matrix.shape[1]
    b2 = b.reshape(1, D)
    eps2 = epsilon.reshape(1, 1)
    return pl.pallas_call(
        _hgmp_kernel,
        in_specs=[
            pl.BlockSpec((N, D), lambda: (0, 0)),
            pl.BlockSpec(memory_space=pl.ANY),
            pl.BlockSpec((D, D), lambda: (0, 0)),
            pl.BlockSpec((1, D), lambda: (0, 0)),
            pl.BlockSpec((1, 1), lambda: (0, 0)),
        ],
        out_specs=pl.BlockSpec((N, D), lambda: (0, 0)),
        out_shape=jax.ShapeDtypeStruct((N, D), jnp.float32),
        scratch_shapes=[
            pltpu.VMEM((N, V), jnp.float32),
            pltpu.SemaphoreType.DMA,
        ],
    )(node_features, incidence_matrix, W, b2, eps2)


# speedup vs baseline: 1.1991x; 1.0384x over previous
import jax
import jax.numpy as jnp
from jax import lax
from jax.experimental import pallas as pl


def _dot_t(a, b):
    return lax.dot_general(a, b, (((0,), (0,)), ((), ())),
                           preferred_element_type=jnp.float32)


def _hgmp_kernel(x_ref, inc_ref, w_ref, b_ref, eps_ref, out_ref):
    x = x_ref[...]
    mask = (inc_ref[...] > 0).astype(jnp.float32)

    sums = _dot_t(mask, x)
    ones = jnp.ones((x.shape[0], 1), dtype=jnp.float32)
    counts = _dot_t(mask, ones)
    vf = sums / jnp.maximum(counts, 1.0)

    svf = jnp.dot(mask, vf, preferred_element_type=jnp.float32)
    combined = (1.0 + eps_ref[0, 0]) * x + svf
    y = lax.dot_general(combined, w_ref[...], (((1,), (1,)), ((), ())),
                        preferred_element_type=jnp.float32) + b_ref[...]
    out_ref[...] = jnp.where(y > 0, y, 0.2 * y)


def kernel(node_features, incidence_matrix, W, b, epsilon):
    N, D = node_features.shape
    b2 = b.reshape(1, D)
    eps2 = epsilon.reshape(1, 1)
    return pl.pallas_call(
        _hgmp_kernel,
        out_shape=jax.ShapeDtypeStruct((N, D), jnp.float32),
    )(node_features, incidence_matrix, W, b2, eps2)


# speedup vs baseline: 1.2002x; 1.0009x over previous
import jax
import jax.numpy as jnp
from jax import lax
from jax.experimental import pallas as pl


def _dot_t(a, b):
    return lax.dot_general(a, b, (((0,), (0,)), ((), ())),
                           preferred_element_type=jnp.float32)


def _hgmp_kernel(x_ref, inc_ref, w_ref, b_ref, eps_ref, out_ref):
    x = x_ref[...]
    mask = (inc_ref[...] > 0).astype(jnp.float32)

    sums = _dot_t(mask, x)
    ones = jnp.ones((x.shape[0], 1), dtype=jnp.float32)
    counts = _dot_t(mask, ones)
    vf = sums / jnp.maximum(counts, 1.0)

    w = w_ref[...]
    vfw = lax.dot_general(vf, w, (((1,), (1,)), ((), ())),
                          preferred_element_type=jnp.float32)
    xw = lax.dot_general(x, w, (((1,), (1,)), ((), ())),
                         preferred_element_type=jnp.float32)
    y = ((1.0 + eps_ref[0, 0]) * xw + b_ref[...]
         + jnp.dot(mask, vfw, preferred_element_type=jnp.float32))
    out_ref[...] = jnp.where(y > 0, y, 0.2 * y)


def kernel(node_features, incidence_matrix, W, b, epsilon):
    N, D = node_features.shape
    b2 = b.reshape(1, D)
    eps2 = epsilon.reshape(1, 1)
    return pl.pallas_call(
        _hgmp_kernel,
        out_shape=jax.ShapeDtypeStruct((N, D), jnp.float32),
    )(node_features, incidence_matrix, W, b2, eps2)
